# Initial kernel scaffold; baseline (speedup 1.0000x reference)
import jax
import jax.numpy as jnp
from jax.experimental import pallas as pl


def kernel(instruction_ids, embed_table):
    raise NotImplementedError("write your pallas kernel here")


# speedup vs baseline: 8.6927x; 8.6927x over previous
import functools

import jax
import jax.numpy as jnp
from jax import lax
from jax.experimental import pallas as pl
from jax.experimental.pallas import tpu as pltpu
from jax.experimental.pallas import tpu_sc as plsc

VOCAB = 100000
HIDDEN = 64
BATCH = 4096
SEQ = 50

NUM_CORES = 2
NUM_SUBCORES = 16
NUM_WORKERS = NUM_CORES * NUM_SUBCORES
ROWS_PER_W = BATCH // NUM_WORKERS
CB = 2
IDS_PER_CHUNK = CB * SEQ
CHUNKS = ROWS_PER_W // CB
LANES = 16
HREGS = HIDDEN // LANES
SCALE = 1.0 / SEQ


def _body(table_hbm, ids_hbm, out_hbm, idx_v, rows0, rows1, out_v, sem0, sem1):
    wid = lax.axis_index("s") * NUM_CORES + lax.axis_index("c")

    pltpu.sync_copy(ids_hbm.at[wid], idx_v)

    def start(j, buf, sem):
        pltpu.async_copy(table_hbm.at[idx_v.at[j]], buf, sem)

    def wait(buf, sem):
        pltpu.make_async_copy(table_hbm.at[idx_v.at[0]], buf, sem).wait()

    def accum(j, buf):
        def step(s, acc):
            return tuple(
                acc[r * HREGS + c] + buf[r * SEQ + s, pl.ds(c * LANES, LANES)]
                for r in range(CB)
                for c in range(HREGS)
            )

        zero = jnp.zeros((LANES,), jnp.float32)
        acc = lax.fori_loop(0, SEQ, step, (zero,) * (CB * HREGS), unroll=5)
        for r in range(CB):
            for c in range(HREGS):
                out_v[j * CB + r, pl.ds(c * LANES, LANES)] = (
                    acc[r * HREGS + c] * SCALE
                )

    start(0, rows0, sem0)

    def outer(i, _):
        j = 2 * i
        start(j + 1, rows1, sem1)
        wait(rows0, sem0)
        accum(j, rows0)
        start(j + 2, rows0, sem0)
        wait(rows1, sem1)
        accum(j + 1, rows1)
        return 0

    lax.fori_loop(0, CHUNKS // 2 - 1, outer, 0)

    start(CHUNKS - 1, rows1, sem1)
    wait(rows0, sem0)
    accum(CHUNKS - 2, rows0)
    wait(rows1, sem1)
    accum(CHUNKS - 1, rows1)

    pltpu.sync_copy(out_v, out_hbm.at[pl.ds(wid * ROWS_PER_W, ROWS_PER_W)])


@jax.jit
def _run(ids3, table):
    mesh = plsc.VectorSubcoreMesh(core_axis_name="c", subcore_axis_name="s")
    f = pl.kernel(
        _body,
        out_type=jax.ShapeDtypeStruct((BATCH, HIDDEN), jnp.float32),
        mesh=mesh,
        scratch_types=[
            pltpu.VMEM((CHUNKS, IDS_PER_CHUNK), jnp.int32),
            pltpu.VMEM((IDS_PER_CHUNK, HIDDEN), jnp.float32),
            pltpu.VMEM((IDS_PER_CHUNK, HIDDEN), jnp.float32),
            pltpu.VMEM((ROWS_PER_W, HIDDEN), jnp.float32),
            pltpu.SemaphoreType.DMA,
            pltpu.SemaphoreType.DMA,
        ],
        compiler_params=pltpu.CompilerParams(use_tc_tiling_on_sc=False),
    )
    return f(table, ids3)


def kernel(instruction_ids, embed_table):
    ids3 = instruction_ids.astype(jnp.int32).reshape(
        NUM_WORKERS, CHUNKS, IDS_PER_CHUNK
    )
    return _run(ids3, embed_table)
